# baseline (speedup 1.0000x reference)
import functools

import jax
import jax.numpy as jnp
from jax import lax
from jax.experimental import pallas as pl
from jax.experimental.pallas import tpu as pltpu
from jax.experimental.pallas import tpu_sc as plsc

NUM_CORES = 2
NUM_SUBCORES = 16
NW = NUM_CORES * NUM_SUBCORES
BATCH = 16384
EMB = 64
BPW = BATCH // NW
CHUNK = 128
NCHUNK = BPW // CHUNK

_GDN = lax.GatherDimensionNumbers(
    offset_dims=(), collapsed_slice_dims=(0,), start_index_map=(0,))


def _permute(x, idx):
    return lax.gather(x, idx[:, None], _GDN, (1,),
                      mode=lax.GatherScatterMode.PROMISE_IN_BOUNDS)


def _sc_body(user_ids, movie_ids, user_table, movie_table, w_hbm, b_hbm,
             out_hbm, idx_u, idx_m, rows_u, rows_m, w_v, b_v, out_v, sem):
    wid = lax.axis_index("s") * NUM_CORES + lax.axis_index("c")
    base = wid * BPW

    for k in range(NCHUNK):
        pltpu.sync_copy(user_ids.at[pl.ds(base + k * CHUNK, CHUNK)], idx_u.at[k])
        pltpu.sync_copy(movie_ids.at[pl.ds(base + k * CHUNK, CHUNK)], idx_m.at[k])
    pltpu.sync_copy(w_hbm, w_v)
    pltpu.sync_copy(b_hbm, b_v)

    copies = []
    for k in range(NCHUNK):
        copies.append(pltpu.async_copy(
            user_table.at[idx_u.at[k]], rows_u.at[pl.ds(k * CHUNK, CHUNK)], sem))
        copies.append(pltpu.async_copy(
            movie_table.at[idx_m.at[k]], rows_m.at[pl.ds(k * CHUNK, CHUNK)], sem))
    for c in copies:
        c.wait()

    w_c = [w_v[pl.ds(c * 16, 16)] for c in range(4)]
    b_vec = b_v[...]
    lane = lax.iota(jnp.int32, 16)
    perms = [lane ^ (1 << k) for k in range(4)]

    def g_body(g, carry):
        out_vec = jnp.zeros((16,), jnp.float32)
        for j in range(16):
            e = g * 16 + j
            ps = None
            for c in range(4):
                u = rows_u[e, pl.ds(c * 16, 16)]
                m = rows_m[e, pl.ds(c * 16, 16)]
                t = u * m * w_c[c]
                ps = t if ps is None else ps + t
            for k in range(4):
                ps = ps + _permute(ps, perms[k])
            out_vec = jnp.where(lane == j, ps, out_vec)
        out_v[pl.ds(g * 16, 16)] = out_vec + b_vec
        return carry

    lax.fori_loop(0, BPW // 16, g_body, 0)

    pltpu.sync_copy(out_v, out_hbm.at[pl.ds(base, BPW)])


@functools.partial(jax.jit, static_argnames=())
def _run(user_ids, movie_ids, user_table, movie_table, w_flat, b16):
    mesh = plsc.VectorSubcoreMesh(
        core_axis_name="c", subcore_axis_name="s",
        num_cores=NUM_CORES, num_subcores=NUM_SUBCORES)
    return pl.kernel(
        _sc_body,
        out_type=jax.ShapeDtypeStruct((BATCH,), jnp.float32),
        mesh=mesh,
        scratch_types=[
            pltpu.VMEM((NCHUNK, CHUNK), jnp.int32),
            pltpu.VMEM((NCHUNK, CHUNK), jnp.int32),
            pltpu.VMEM((BPW, EMB), jnp.float32),
            pltpu.VMEM((BPW, EMB), jnp.float32),
            pltpu.VMEM((EMB,), jnp.float32),
            pltpu.VMEM((16,), jnp.float32),
            pltpu.VMEM((BPW,), jnp.float32),
            pltpu.SemaphoreType.DMA,
        ],
        compiler_params=pltpu.CompilerParams(use_tc_tiling_on_sc=False),
    )(user_ids, movie_ids, user_table, movie_table, w_flat, b16)


def kernel(user_ids, movie_ids, user_table, movie_table, W, b):
    w_flat = W.reshape(EMB).astype(jnp.float32)
    b16 = jnp.broadcast_to(b.astype(jnp.float32), (16,))
    return _run(user_ids.astype(jnp.int32), movie_ids.astype(jnp.int32),
                user_table, movie_table, w_flat, b16)


# speedup vs baseline: 2.4045x; 2.4045x over previous
import functools

import jax
import jax.numpy as jnp
from jax import lax
from jax.experimental import pallas as pl
from jax.experimental.pallas import tpu as pltpu
from jax.experimental.pallas import tpu_sc as plsc

NUM_CORES = 2
NUM_SUBCORES = 16
NW = NUM_CORES * NUM_SUBCORES
BATCH = 16384
EMB = 64
BPW = BATCH // NW
CH = 256

_GDN = lax.GatherDimensionNumbers(
    offset_dims=(), collapsed_slice_dims=(0,), start_index_map=(0,))


def _permute(x, idx):
    return lax.gather(x, idx[:, None], _GDN, (1,),
                      mode=lax.GatherScatterMode.PROMISE_IN_BOUNDS)


def _sc_body(user_ids, movie_ids, ut3, mt3, w_hbm, b_hbm,
             out_hbm, ids_v, rows_u, rows_m, w_v, b_v,
             out_v, sem_u, sem_m):
    wid = lax.axis_index("s") * NUM_CORES + lax.axis_index("c")
    base = wid * BPW

    pltpu.sync_copy(user_ids.at[pl.ds(base, BPW)], ids_v.at[0])
    pltpu.sync_copy(movie_ids.at[pl.ds(base, BPW)], ids_v.at[1])
    pltpu.sync_copy(w_hbm, w_v)
    pltpu.sync_copy(b_hbm, b_v)

    w_c = [w_v[pl.ds(c * 16, 16)] for c in range(4)]
    b_vec = b_v[...]
    lane = lax.iota(jnp.int32, 16)
    perms = [lane ^ (1 << k) for k in range(4)]

    def chunk(k, carry):
        coff = k * CH

        def fire(g, c2):
            uvec = ids_v[0, pl.ds(coff + g * 16, 16)]
            mvec = ids_v[1, pl.ds(coff + g * 16, 16)]
            for j in range(16):
                e = g * 16 + j
                uid = lax.index_in_dim(uvec, j, 0, keepdims=False)
                pltpu.async_copy(ut3.at[uid // 8, uid % 8], rows_u.at[e],
                                 sem_u)
                mid = lax.index_in_dim(mvec, j, 0, keepdims=False)
                pltpu.async_copy(mt3.at[mid // 8, mid % 8], rows_m.at[e],
                                 sem_m)
            return c2

        lax.fori_loop(0, CH // 16, fire, 0)

        def drain(e, c2):
            pltpu.make_async_copy(ut3.at[0, 0], rows_u.at[e], sem_u).wait()
            pltpu.make_async_copy(mt3.at[0, 0], rows_m.at[e], sem_m).wait()
            return c2

        lax.fori_loop(0, CH, drain, 0)

        def g_body(g, c2):
            out_vec = jnp.zeros((16,), jnp.float32)
            for j in range(16):
                e = g * 16 + j
                ps = None
                for c in range(4):
                    u = rows_u[e, pl.ds(c * 16, 16)]
                    m = rows_m[e, pl.ds(c * 16, 16)]
                    t = u * m * w_c[c]
                    ps = t if ps is None else ps + t
                for k2 in range(4):
                    ps = ps + _permute(ps, perms[k2])
                out_vec = jnp.where(lane == j, ps, out_vec)
            out_v[pl.ds(coff + g * 16, 16)] = out_vec + b_vec
            return c2

        lax.fori_loop(0, CH // 16, g_body, 0)
        return carry

    lax.fori_loop(0, BPW // CH, chunk, 0)

    pltpu.sync_copy(out_v, out_hbm.at[pl.ds(base, BPW)])


@jax.jit
def _run(user_ids, movie_ids, ut3, mt3, w_flat, b16):
    mesh = plsc.VectorSubcoreMesh(
        core_axis_name="c", subcore_axis_name="s",
        num_cores=NUM_CORES, num_subcores=NUM_SUBCORES)
    return pl.kernel(
        _sc_body,
        out_type=jax.ShapeDtypeStruct((BATCH,), jnp.float32),
        mesh=mesh,
        scratch_types=[
            pltpu.VMEM((2, BPW), jnp.int32),
            pltpu.VMEM((CH, EMB), jnp.float32),
            pltpu.VMEM((CH, EMB), jnp.float32),
            pltpu.VMEM((EMB,), jnp.float32),
            pltpu.VMEM((16,), jnp.float32),
            pltpu.VMEM((BPW,), jnp.float32),
            pltpu.SemaphoreType.DMA,
            pltpu.SemaphoreType.DMA,
        ],
    )(user_ids, movie_ids, ut3, mt3, w_flat, b16)


def kernel(user_ids, movie_ids, user_table, movie_table, W, b):
    ut3 = user_table.reshape(user_table.shape[0] // 8, 8, EMB)
    mt3 = movie_table.reshape(movie_table.shape[0] // 8, 8, EMB)
    w_flat = W.reshape(EMB).astype(jnp.float32)
    b16 = jnp.broadcast_to(b.astype(jnp.float32), (16,))
    return _run(user_ids.astype(jnp.int32), movie_ids.astype(jnp.int32),
                ut3, mt3, w_flat, b16)
